# Initial kernel scaffold; baseline (speedup 1.0000x reference)
import jax
import jax.numpy as jnp
from jax.experimental import pallas as pl


def kernel(new_xyz, xyz, feature, use_xyz):
    raise NotImplementedError("write your pallas kernel here")


# speedup vs baseline: 1.0001x; 1.0001x over previous
import functools

import jax
import jax.numpy as jnp
from jax.experimental import pallas as pl

RADIUS = 0.2
K = 64


def _cand_body(nxyz_t_ref, xyz_ref, cand_ref):
    nx = nxyz_t_ref[0]
    p = xyz_ref[0]
    qx = nx[:, 0:1]
    qy = nx[:, 1:2]
    qz = nx[:, 2:3]
    px = p[0:1, :]
    py = p[1:2, :]
    pz = p[2:3, :]
    cross = jax.lax.dot_general(nx, p, (((1,), (0,)), ((), ())),
                                preferred_element_type=jnp.float32)
    q2 = (qx * qx + qy * qy) + qz * qz
    p2 = (px * px + py * py) + pz * pz
    d2 = (q2 + p2) - 2.0 * cross
    n = cand_ref.shape[2]
    iota = jax.lax.broadcasted_iota(jnp.int32, d2.shape, 1)
    r2 = jnp.float32(RADIUS * RADIUS)
    cand_ref[0] = jnp.where(d2 < r2, iota, n)


def _ball_query_cand(new_xyz, xyz):
    B, _, M = new_xyz.shape
    N = xyz.shape[2]
    Mb = 256
    nxyz_t = jnp.transpose(new_xyz, (0, 2, 1))
    grid = (B, M // Mb)
    return pl.pallas_call(
        _cand_body,
        grid=grid,
        in_specs=[
            pl.BlockSpec((1, Mb, 3), lambda b, i: (b, i, 0)),
            pl.BlockSpec((1, 3, N), lambda b, i: (b, 0, 0)),
        ],
        out_specs=pl.BlockSpec((1, Mb, N), lambda b, i: (b, i, 0)),
        out_shape=jax.ShapeDtypeStruct((B, M, N), jnp.int32),
    )(nxyz_t, xyz)


def _group(x, idx):
    B, C, N = x.shape
    _, M, Kk = idx.shape
    flat = idx.reshape(B, 1, M * Kk)
    g = jnp.take_along_axis(x, jnp.broadcast_to(flat, (B, C, M * Kk)), axis=2)
    return g.reshape(B, C, M, Kk)


def kernel(new_xyz, xyz, feature, use_xyz):
    B, _, M = new_xyz.shape
    N = xyz.shape[2]
    cand = _ball_query_cand(new_xyz, xyz)
    sidx = jnp.sort(cand, axis=-1)[:, :, :K]
    first = sidx[:, :, :1]
    first = jnp.where(first == N, 0, first)
    idx = jnp.where(sidx == N, first, sidx).astype(jnp.int32)
    group_xyz = _group(xyz, idx) - new_xyz[:, :, :, None]
    gate = (jnp.asarray(use_xyz) != 0).astype(group_xyz.dtype)
    group_feature = jnp.concatenate([_group(feature, idx), group_xyz * gate], axis=1)
    return (group_feature, group_xyz)


# speedup vs baseline: 1.0102x; 1.0101x over previous
import functools

import jax
import jax.numpy as jnp
from jax import lax
from jax.experimental import pallas as pl
from jax.experimental.pallas import tpu as pltpu
from jax.experimental.pallas import tpu_sc as plsc

RADIUS = 0.2
K = 64
NC = 2
NS = 16
L = 16


def _cand_body(nxyz_t_ref, xyz_ref, cand_ref):
    nx = nxyz_t_ref[0]
    p = xyz_ref[0]
    qx = nx[:, 0:1]
    qy = nx[:, 1:2]
    qz = nx[:, 2:3]
    px = p[0:1, :]
    py = p[1:2, :]
    pz = p[2:3, :]
    cross = jax.lax.dot_general(nx, p, (((1,), (0,)), ((), ())),
                                preferred_element_type=jnp.float32)
    q2 = (qx * qx + qy * qy) + qz * qz
    p2 = (px * px + py * py) + pz * pz
    d2 = (q2 + p2) - 2.0 * cross
    iota = jax.lax.broadcasted_iota(jnp.int32, d2.shape, 1)
    r2 = jnp.float32(RADIUS * RADIUS)
    cand_ref[0] = jnp.where(d2 < r2, iota, -1)


def _ball_query_cand(new_xyz, xyz):
    B, _, M = new_xyz.shape
    N = xyz.shape[2]
    Mb = 256
    nxyz_t = jnp.transpose(new_xyz, (0, 2, 1))
    grid = (B, M // Mb)
    return pl.pallas_call(
        _cand_body,
        grid=grid,
        in_specs=[
            pl.BlockSpec((1, Mb, 3), lambda b, i: (b, i, 0)),
            pl.BlockSpec((1, 3, N), lambda b, i: (b, 0, 0)),
        ],
        out_specs=pl.BlockSpec((1, Mb, N), lambda b, i: (b, i, 0)),
        out_shape=jax.ShapeDtypeStruct((B, M, N), jnp.int32),
    )(nxyz_t, xyz)


def _make_select(rows, n):
    ngroups = n // L
    rpw = rows // (NC * NS)
    npairs = rpw // 2
    mesh = plsc.VectorSubcoreMesh(core_axis_name="c", subcore_axis_name="s")

    @functools.partial(
        pl.kernel,
        out_type=jax.ShapeDtypeStruct((rows * K,), jnp.int32),
        mesh=mesh,
        compiler_params=pltpu.CompilerParams(needs_layout_passes=False),
        scratch_types=[
            pltpu.VMEM((n,), jnp.int32),
            pltpu.VMEM((n,), jnp.int32),
            pltpu.VMEM((K + L,), jnp.int32),
            pltpu.VMEM((rpw * K,), jnp.int32),
            pltpu.SemaphoreType.DMA,
            pltpu.SemaphoreType.DMA,
        ],
    )
    def select(cand_hbm, idx_hbm, cbuf0, cbuf1, rowbuf, outbuf, sem0, sem1):
        cid = lax.axis_index("c")
        sid = lax.axis_index("s")
        wid = sid * NC + cid
        base = wid * rpw

        zeros16 = jnp.zeros((L,), jnp.int32)

        def scan_row(cbuf, r_local):
            def body(w, ptrv):
                v = cbuf[pl.ds(w * L, L)]
                msk = v >= 0
                cum = plsc.cumsum(msk.astype(jnp.int32))
                pos = ptrv + cum - 1
                mske = jnp.logical_and(msk, pos < K)
                posc = jnp.minimum(pos, K + L - 1)
                plsc.store_scatter(rowbuf, [posc], v, mask=mske)
                return ptrv + plsc.all_reduce_population_count(msk)

            ptrv = lax.fori_loop(0, ngroups, body, zeros16)
            v0 = rowbuf[pl.ds(0, L)]
            lanes0 = lax.iota(jnp.int32, L)
            firstv = plsc.cummax(jnp.where(lanes0 == 0, v0, jnp.int32(-2147483648)))
            fvec = jnp.where(ptrv > 0, firstv, 0)
            obase = r_local * K
            for g in range(K // L):
                cur = rowbuf[pl.ds(g * L, L)]
                lanes = lax.iota(jnp.int32, L) + (g * L)
                outbuf[pl.ds(obase + g * L, L)] = jnp.where(lanes < ptrv, cur, fvec)

        pltpu.async_copy(cand_hbm.at[base], cbuf0, sem0)

        def pair(j, _):
            r0 = base + 2 * j
            pltpu.async_copy(cand_hbm.at[r0 + 1], cbuf1, sem1)
            pltpu.make_async_copy(cand_hbm.at[r0], cbuf0, sem0).wait()
            scan_row(cbuf0, 2 * j)

            @pl.when(j < npairs - 1)
            def _():
                pltpu.async_copy(cand_hbm.at[r0 + 2], cbuf0, sem0)

            pltpu.make_async_copy(cand_hbm.at[r0 + 1], cbuf1, sem1).wait()
            scan_row(cbuf1, 2 * j + 1)
            return 0

        lax.fori_loop(0, npairs, pair, 0)
        pltpu.sync_copy(outbuf, idx_hbm.at[pl.ds(base * K, rpw * K)])

    return select


def _group(x, idx):
    B, C, N = x.shape
    _, M, Kk = idx.shape
    flat = idx.reshape(B, 1, M * Kk)
    g = jnp.take_along_axis(x, jnp.broadcast_to(flat, (B, C, M * Kk)), axis=2)
    return g.reshape(B, C, M, Kk)


def kernel(new_xyz, xyz, feature, use_xyz):
    B, _, M = new_xyz.shape
    N = xyz.shape[2]
    cand = _ball_query_cand(new_xyz, xyz)
    select = _make_select(B * M, N)
    idx = select(cand.reshape(B * M, N)).reshape(B, M, K)
    group_xyz = _group(xyz, idx) - new_xyz[:, :, :, None]
    gate = (jnp.asarray(use_xyz) != 0).astype(group_xyz.dtype)
    group_feature = jnp.concatenate([_group(feature, idx), group_xyz * gate], axis=1)
    return (group_feature, group_xyz)


# speedup vs baseline: 196.8416x; 194.8538x over previous
import functools

import jax
import jax.numpy as jnp
from jax import lax
from jax.experimental import pallas as pl
from jax.experimental.pallas import tpu as pltpu
from jax.experimental.pallas import tpu_sc as plsc

RADIUS = 0.2
K = 64
NC = 2
NS = 16
L = 16


def _cand_body(nxyz_t_ref, xyz_ref, cand_ref):
    nx = nxyz_t_ref[0]
    p = xyz_ref[0]
    qx = nx[:, 0:1]
    qy = nx[:, 1:2]
    qz = nx[:, 2:3]
    px = p[0:1, :]
    py = p[1:2, :]
    pz = p[2:3, :]
    cross = jax.lax.dot_general(nx, p, (((1,), (0,)), ((), ())),
                                preferred_element_type=jnp.float32)
    q2 = (qx * qx + qy * qy) + qz * qz
    p2 = (px * px + py * py) + pz * pz
    d2 = (q2 + p2) - 2.0 * cross
    iota = jax.lax.broadcasted_iota(jnp.int32, d2.shape, 1)
    r2 = jnp.float32(RADIUS * RADIUS)
    cand_ref[0] = jnp.where(d2 < r2, iota, -1)


def _ball_query_cand(new_xyz, xyz):
    B, _, M = new_xyz.shape
    N = xyz.shape[2]
    Mb = 256
    nxyz_t = jnp.transpose(new_xyz, (0, 2, 1))
    grid = (B, M // Mb)
    return pl.pallas_call(
        _cand_body,
        grid=grid,
        in_specs=[
            pl.BlockSpec((1, Mb, 3), lambda b, i: (b, i, 0)),
            pl.BlockSpec((1, 3, N), lambda b, i: (b, 0, 0)),
        ],
        out_specs=pl.BlockSpec((1, Mb, N), lambda b, i: (b, i, 0)),
        out_shape=jax.ShapeDtypeStruct((B, M, N), jnp.int32),
    )(nxyz_t, xyz)


def _make_select(rows, n):
    ngroups = n // L
    rpw = rows // (NC * NS)
    npairs = rpw // 2
    mesh = plsc.VectorSubcoreMesh(core_axis_name="c", subcore_axis_name="s")

    @functools.partial(
        pl.kernel,
        out_type=jax.ShapeDtypeStruct((rows * K,), jnp.int32),
        mesh=mesh,
        compiler_params=pltpu.CompilerParams(needs_layout_passes=False),
        scratch_types=[
            pltpu.VMEM((n,), jnp.int32),
            pltpu.VMEM((n,), jnp.int32),
            pltpu.VMEM((K + L,), jnp.int32),
            pltpu.VMEM((rpw * K,), jnp.int32),
            pltpu.SemaphoreType.DMA,
            pltpu.SemaphoreType.DMA,
        ],
    )
    def select(cand_hbm, idx_hbm, cbuf0, cbuf1, rowbuf, outbuf, sem0, sem1):
        cid = lax.axis_index("c")
        sid = lax.axis_index("s")
        wid = sid * NC + cid
        base = wid * rpw

        zeros16 = jnp.zeros((L,), jnp.int32)

        def scan_row(cbuf, r_local):
            def body(w, ptrv):
                v = cbuf[pl.ds(w * L, L)]
                msk = v >= 0
                cum = plsc.cumsum(msk.astype(jnp.int32))
                pos = ptrv + cum - 1
                mske = jnp.logical_and(msk, pos < K)
                posc = jnp.minimum(pos, K + L - 1)
                plsc.store_scatter(rowbuf, [posc], v, mask=mske)
                return ptrv + plsc.all_reduce_population_count(msk)

            ptrv = lax.fori_loop(0, ngroups, body, zeros16)
            v0 = rowbuf[pl.ds(0, L)]
            lanes0 = lax.iota(jnp.int32, L)
            firstv = plsc.cummax(jnp.where(lanes0 == 0, v0, jnp.int32(-2147483648)))
            fvec = jnp.where(ptrv > 0, firstv, 0)
            obase = r_local * K
            for g in range(K // L):
                cur = rowbuf[pl.ds(g * L, L)]
                lanes = lax.iota(jnp.int32, L) + (g * L)
                outbuf[pl.ds(obase + g * L, L)] = jnp.where(lanes < ptrv, cur, fvec)

        pltpu.async_copy(cand_hbm.at[base], cbuf0, sem0)

        def pair(j, _):
            r0 = base + 2 * j
            pltpu.async_copy(cand_hbm.at[r0 + 1], cbuf1, sem1)
            pltpu.make_async_copy(cand_hbm.at[r0], cbuf0, sem0).wait()
            scan_row(cbuf0, 2 * j)

            @pl.when(j < npairs - 1)
            def _():
                pltpu.async_copy(cand_hbm.at[r0 + 2], cbuf0, sem0)

            pltpu.make_async_copy(cand_hbm.at[r0 + 1], cbuf1, sem1).wait()
            scan_row(cbuf1, 2 * j + 1)
            return 0

        lax.fori_loop(0, npairs, pair, 0)
        pltpu.sync_copy(outbuf, idx_hbm.at[pl.ds(base * K, rpw * K)])

    return select


def _make_gather(B, C, M, N):
    MK = M * K
    HALF = MK // 2
    CTOT = C + 6
    NSLAB = 2 * B
    SPS = (NC * NS) // NSLAB
    CPS = (CTOT + SPS - 1) // SPS
    CHUNK = 4096
    NCHUNK = HALF // CHUNK
    NGRP = CHUNK // L
    mesh = plsc.VectorSubcoreMesh(core_axis_name="c", subcore_axis_name="s")

    @functools.partial(
        pl.kernel,
        out_type=(
            jax.ShapeDtypeStruct((B, C + 3, MK), jnp.float32),
            jax.ShapeDtypeStruct((B, 3, MK), jnp.float32),
        ),
        mesh=mesh,
        compiler_params=pltpu.CompilerParams(needs_layout_passes=False),
        scratch_types=[
            pltpu.VMEM((HALF,), jnp.int32),
            pltpu.VMEM((N,), jnp.float32),
            pltpu.VMEM((CHUNK,), jnp.float32),
            pltpu.VMEM((CHUNK,), jnp.float32),
            pltpu.VMEM((CHUNK,), jnp.float32),
            pltpu.SemaphoreType.DMA,
            pltpu.SemaphoreType.DMA,
        ],
    )
    def gather(idx_hbm, feat_hbm, xyzg_hbm, xyz_hbm, nxeg_hbm, nxe_hbm,
               ofeat_hbm, ogxyz_hbm, ibuf, tbl, obuf0, obuf1, sbuf,
               sem0, sem1):
        cid = lax.axis_index("c")
        sid = lax.axis_index("s")
        wid = sid * NC + cid
        slab = wid // SPS
        lane = wid % SPS
        b = slab // 2
        h = slab % 2
        slab_off = h * HALF

        pltpu.sync_copy(idx_hbm.at[pl.ds(b * MK + slab_off, HALF)], ibuf)

        def do_channel(ci, _):
            c_glob = lane * CPS + ci

            @pl.when(c_glob < CTOT)
            def _():
                is_feat = c_glob < C
                is_fxyz = jnp.logical_and(c_glob >= C, c_glob < C + 3)
                is_gxyz = c_glob >= C + 3
                cf = jnp.minimum(c_glob, C - 1)
                cx = jnp.clip(c_glob - C, 0, 2)
                cg = jnp.clip(c_glob - (C + 3), 0, 2)

                @pl.when(is_feat)
                def _():
                    pltpu.sync_copy(feat_hbm.at[b, cf], tbl)

                @pl.when(is_fxyz)
                def _():
                    pltpu.sync_copy(xyzg_hbm.at[b, cx], tbl)

                @pl.when(is_gxyz)
                def _():
                    pltpu.sync_copy(xyz_hbm.at[b, cg], tbl)

                def fill(obuf, ch):
                    base = ch * CHUNK

                    @pl.when(jnp.logical_not(is_feat))
                    def _():
                        @pl.when(is_fxyz)
                        def _():
                            pltpu.sync_copy(
                                nxeg_hbm.at[b, cx, pl.ds(slab_off + base, CHUNK)], sbuf)

                        @pl.when(is_gxyz)
                        def _():
                            pltpu.sync_copy(
                                nxe_hbm.at[b, cg, pl.ds(slab_off + base, CHUNK)], sbuf)

                    def grp(i, _):
                        for u in range(4):
                            o = (i * 4 + u) * L
                            iv = ibuf[pl.ds(base + o, L)]
                            g = plsc.load_gather(tbl, [iv])
                            obuf[pl.ds(o, L)] = g
                        return 0

                    def grp_sub(i, _):
                        for u in range(4):
                            o = (i * 4 + u) * L
                            iv = ibuf[pl.ds(base + o, L)]
                            g = plsc.load_gather(tbl, [iv])
                            obuf[pl.ds(o, L)] = g - sbuf[pl.ds(o, L)]
                        return 0

                    @pl.when(is_feat)
                    def _():
                        lax.fori_loop(0, NGRP // 4, grp, 0)

                    @pl.when(jnp.logical_not(is_feat))
                    def _():
                        lax.fori_loop(0, NGRP // 4, grp_sub, 0)

                def flush(obuf, ch, sem):
                    base = ch * CHUNK
                    dst_off = slab_off + base

                    @pl.when(jnp.logical_not(is_gxyz))
                    def _():
                        pltpu.async_copy(
                            obuf, ofeat_hbm.at[b, jnp.minimum(c_glob, C + 2),
                                               pl.ds(dst_off, CHUNK)], sem)

                    @pl.when(is_gxyz)
                    def _():
                        pltpu.async_copy(
                            obuf, ogxyz_hbm.at[b, cg, pl.ds(dst_off, CHUNK)], sem)

                def wait(obuf, ch, sem):
                    base = ch * CHUNK
                    dst_off = slab_off + base

                    @pl.when(jnp.logical_not(is_gxyz))
                    def _():
                        pltpu.make_async_copy(
                            obuf, ofeat_hbm.at[b, jnp.minimum(c_glob, C + 2),
                                               pl.ds(dst_off, CHUNK)], sem).wait()

                    @pl.when(is_gxyz)
                    def _():
                        pltpu.make_async_copy(
                            obuf, ogxyz_hbm.at[b, cg, pl.ds(dst_off, CHUNK)], sem).wait()

                def chunk_pair(j, _):
                    fill(obuf0, 2 * j)
                    flush(obuf0, 2 * j, sem0)
                    fill(obuf1, 2 * j + 1)
                    flush(obuf1, 2 * j + 1, sem1)
                    wait(obuf0, 2 * j, sem0)
                    wait(obuf1, 2 * j + 1, sem1)
                    return 0

                lax.fori_loop(0, NCHUNK // 2, chunk_pair, 0)

            return 0

        lax.fori_loop(0, CPS, do_channel, 0)

    return gather


def kernel(new_xyz, xyz, feature, use_xyz):
    B, _, M = new_xyz.shape
    C = feature.shape[1]
    N = xyz.shape[2]
    cand = _ball_query_cand(new_xyz, xyz)
    select = _make_select(B * M, N)
    idx = select(cand.reshape(B * M, N))
    gate = (jnp.asarray(use_xyz) != 0).astype(jnp.float32)
    xyz_g = xyz * gate
    nxe = jnp.broadcast_to(new_xyz[..., None], (B, 3, M, K)).reshape(B, 3, M * K)
    nxe_g = nxe * gate
    gather = _make_gather(B, C, M, N)
    ofeat, ogxyz = gather(idx, feature, xyz_g, xyz, nxe_g, nxe)
    group_feature = ofeat.reshape(B, C + 3, M, K)
    group_xyz = ogxyz.reshape(B, 3, M, K)
    return (group_feature, group_xyz)


# speedup vs baseline: 197.9750x; 1.0058x over previous
import functools

import jax
import jax.numpy as jnp
from jax import lax
from jax.experimental import pallas as pl
from jax.experimental.pallas import tpu as pltpu
from jax.experimental.pallas import tpu_sc as plsc

RADIUS = 0.2
K = 64
NC = 2
NS = 16
L = 16


def _cand_body(nxyz_t_ref, xyz_ref, cand_ref):
    nx = nxyz_t_ref[0]
    p = xyz_ref[0]
    qx = nx[:, 0:1]
    qy = nx[:, 1:2]
    qz = nx[:, 2:3]
    px = p[0:1, :]
    py = p[1:2, :]
    pz = p[2:3, :]
    cross = jax.lax.dot_general(nx, p, (((1,), (0,)), ((), ())),
                                preferred_element_type=jnp.float32)
    q2 = (qx * qx + qy * qy) + qz * qz
    p2 = (px * px + py * py) + pz * pz
    d2 = (q2 + p2) - 2.0 * cross
    iota = jax.lax.broadcasted_iota(jnp.int32, d2.shape, 1)
    r2 = jnp.float32(RADIUS * RADIUS)
    cand_ref[0] = jnp.where(d2 < r2, iota, -1)


def _ball_query_cand(new_xyz, xyz):
    B, _, M = new_xyz.shape
    N = xyz.shape[2]
    Mb = 256
    nxyz_t = jnp.transpose(new_xyz, (0, 2, 1))
    grid = (B, M // Mb)
    return pl.pallas_call(
        _cand_body,
        grid=grid,
        in_specs=[
            pl.BlockSpec((1, Mb, 3), lambda b, i: (b, i, 0)),
            pl.BlockSpec((1, 3, N), lambda b, i: (b, 0, 0)),
        ],
        out_specs=pl.BlockSpec((1, Mb, N), lambda b, i: (b, i, 0)),
        out_shape=jax.ShapeDtypeStruct((B, M, N), jnp.int32),
    )(nxyz_t, xyz)


def _make_select(B, M, n):
    rows = B * M
    ngroups = n // L
    rpw = rows // (NC * NS)
    npairs = rpw // 2
    UNROLL = 8
    mesh = plsc.VectorSubcoreMesh(core_axis_name="c", subcore_axis_name="s")

    @functools.partial(
        pl.kernel,
        out_type=jax.ShapeDtypeStruct((rows * K,), jnp.int32),
        mesh=mesh,
        compiler_params=pltpu.CompilerParams(needs_layout_passes=False),
        scratch_types=[
            pltpu.VMEM((n,), jnp.int32),
            pltpu.VMEM((n,), jnp.int32),
            pltpu.VMEM((n + L,), jnp.int32),
            pltpu.VMEM((rpw * K,), jnp.int32),
            pltpu.SemaphoreType.DMA,
            pltpu.SemaphoreType.DMA,
        ],
    )
    def select(cand_hbm, idx_hbm, cbuf0, cbuf1, rowbuf, outbuf, sem0, sem1):
        cid = lax.axis_index("c")
        sid = lax.axis_index("s")
        wid = sid * NC + cid
        base = wid * rpw

        zeros16 = jnp.zeros((L,), jnp.int32)

        def start_row_copy(r, cbuf, sem):
            pltpu.async_copy(cand_hbm.at[r // M, r % M], cbuf, sem)

        def wait_row_copy(r, cbuf, sem):
            pltpu.make_async_copy(cand_hbm.at[r // M, r % M], cbuf, sem).wait()

        def scan_row(cbuf, r_local):
            def body(w, ptrv):
                for u in range(UNROLL):
                    v = cbuf[pl.ds((w * UNROLL + u) * L, L)]
                    msk = v >= 0
                    cum = plsc.cumsum(msk.astype(jnp.int32))
                    pos = ptrv + cum - 1
                    mske = jnp.logical_and(msk, pos < K)
                    plsc.store_scatter(rowbuf, [pos], v, mask=mske)
                    ptrv = ptrv + plsc.all_reduce_population_count(msk)
                return ptrv

            ptrv = lax.fori_loop(0, ngroups // UNROLL, body, zeros16)
            v0 = rowbuf[pl.ds(0, L)]
            lanes0 = lax.iota(jnp.int32, L)
            firstv = plsc.cummax(jnp.where(lanes0 == 0, v0, jnp.int32(-2147483648)))
            fvec = jnp.where(ptrv > 0, firstv, 0)
            obase = r_local * K
            for g in range(K // L):
                cur = rowbuf[pl.ds(g * L, L)]
                lanes = lax.iota(jnp.int32, L) + (g * L)
                outbuf[pl.ds(obase + g * L, L)] = jnp.where(lanes < ptrv, cur, fvec)

        start_row_copy(base, cbuf0, sem0)

        def pair(j, _):
            r0 = base + 2 * j
            start_row_copy(r0 + 1, cbuf1, sem1)
            wait_row_copy(r0, cbuf0, sem0)
            scan_row(cbuf0, 2 * j)

            @pl.when(j < npairs - 1)
            def _():
                start_row_copy(r0 + 2, cbuf0, sem0)

            wait_row_copy(r0 + 1, cbuf1, sem1)
            scan_row(cbuf1, 2 * j + 1)
            return 0

        lax.fori_loop(0, npairs, pair, 0)
        pltpu.sync_copy(outbuf, idx_hbm.at[pl.ds(base * K, rpw * K)])

    return select


def _make_gather(B, C, M, N):
    MK = M * K
    HALF = MK // 2
    CTOT = C + 6
    NSLAB = 2 * B
    SPS = (NC * NS) // NSLAB
    CPS = (CTOT + SPS - 1) // SPS
    CHUNK = 4096
    NCHUNK = HALF // CHUNK
    NGRP = CHUNK // L
    NBUF = 4
    mesh = plsc.VectorSubcoreMesh(core_axis_name="c", subcore_axis_name="s")

    @functools.partial(
        pl.kernel,
        out_type=(
            jax.ShapeDtypeStruct((B, C + 3, MK), jnp.float32),
            jax.ShapeDtypeStruct((B, 3, MK), jnp.float32),
        ),
        mesh=mesh,
        compiler_params=pltpu.CompilerParams(needs_layout_passes=False),
        scratch_types=[
            pltpu.VMEM((HALF,), jnp.int32),
            pltpu.VMEM((N,), jnp.float32),
            pltpu.VMEM((M,), jnp.float32),
            [pltpu.VMEM((CHUNK,), jnp.float32) for _ in range(NBUF)],
            [pltpu.SemaphoreType.DMA for _ in range(NBUF)],
            pltpu.SemaphoreType.DMA,
        ],
    )
    def gather(idx_hbm, feat_hbm, xyzg_hbm, xyz_hbm, nxg_hbm, nx_hbm,
               ofeat_hbm, ogxyz_hbm, ibuf, tbl, nxtbl, obufs, osems, sem0):
        cid = lax.axis_index("c")
        sid = lax.axis_index("s")
        wid = sid * NC + cid
        slab = wid // SPS
        lane = wid % SPS
        b = slab // 2
        h = slab % 2
        slab_off = h * HALF
        lanes0 = lax.iota(jnp.int32, L)

        pltpu.sync_copy(idx_hbm.at[pl.ds(b * MK + slab_off, HALF)], ibuf)

        def do_channel(ci, _):
            c_glob = lane * CPS + ci

            @pl.when(c_glob < CTOT)
            def _():
                is_feat = c_glob < C
                is_fxyz = jnp.logical_and(c_glob >= C, c_glob < C + 3)
                is_gxyz = c_glob >= C + 3
                cf = jnp.minimum(c_glob, C - 1)
                cx = jnp.clip(c_glob - C, 0, 2)
                cg = jnp.clip(c_glob - (C + 3), 0, 2)
                oc = jnp.minimum(c_glob, C + 2)

                @pl.when(is_feat)
                def _():
                    pltpu.sync_copy(feat_hbm.at[b, cf], tbl)

                @pl.when(is_fxyz)
                def _():
                    pltpu.sync_copy(xyzg_hbm.at[b, cx], tbl)
                    pltpu.sync_copy(nxg_hbm.at[b, cx], nxtbl)

                @pl.when(is_gxyz)
                def _():
                    pltpu.sync_copy(xyz_hbm.at[b, cg], tbl)
                    pltpu.sync_copy(nx_hbm.at[b, cg], nxtbl)

                def fill(obuf, ch):
                    base = ch * CHUNK

                    def grp(i, _):
                        for u in range(4):
                            o = (i * 4 + u) * L
                            iv = ibuf[pl.ds(base + o, L)]
                            g = plsc.load_gather(tbl, [iv])
                            obuf[pl.ds(o, L)] = g
                        return 0

                    def grp_sub(i, _):
                        for u in range(4):
                            o = (i * 4 + u) * L
                            iv = ibuf[pl.ds(base + o, L)]
                            g = plsc.load_gather(tbl, [iv])
                            mv = (slab_off + base + o + lanes0) // K
                            nxv = plsc.load_gather(nxtbl, [mv])
                            obuf[pl.ds(o, L)] = g - nxv
                        return 0

                    @pl.when(is_feat)
                    def _():
                        lax.fori_loop(0, NGRP // 4, grp, 0)

                    @pl.when(jnp.logical_not(is_feat))
                    def _():
                        lax.fori_loop(0, NGRP // 4, grp_sub, 0)

                def flush(obuf, ch, sem):
                    dst_off = slab_off + ch * CHUNK

                    @pl.when(jnp.logical_not(is_gxyz))
                    def _():
                        pltpu.async_copy(
                            obuf, ofeat_hbm.at[b, oc, pl.ds(dst_off, CHUNK)], sem)

                    @pl.when(is_gxyz)
                    def _():
                        pltpu.async_copy(
                            obuf, ogxyz_hbm.at[b, cg, pl.ds(dst_off, CHUNK)], sem)

                def wait_flush(obuf, ch, sem):
                    dst_off = slab_off + ch * CHUNK

                    @pl.when(jnp.logical_not(is_gxyz))
                    def _():
                        pltpu.make_async_copy(
                            obuf, ofeat_hbm.at[b, oc, pl.ds(dst_off, CHUNK)],
                            sem).wait()

                    @pl.when(is_gxyz)
                    def _():
                        pltpu.make_async_copy(
                            obuf, ogxyz_hbm.at[b, cg, pl.ds(dst_off, CHUNK)],
                            sem).wait()

                def ring(j, _):
                    for s in range(NBUF):
                        ch = j * NBUF + s

                        @pl.when(j > 0)
                        def _():
                            wait_flush(obufs[s], ch - NBUF, osems[s])

                        fill(obufs[s], ch)
                        flush(obufs[s], ch, osems[s])
                    return 0

                lax.fori_loop(0, NCHUNK // NBUF, ring, 0)
                for s in range(NBUF):
                    wait_flush(obufs[s], NCHUNK - NBUF + s, osems[s])

            return 0

        lax.fori_loop(0, CPS, do_channel, 0)

    return gather


def kernel(new_xyz, xyz, feature, use_xyz):
    B, _, M = new_xyz.shape
    C = feature.shape[1]
    N = xyz.shape[2]
    cand = _ball_query_cand(new_xyz, xyz)
    select = _make_select(B, M, N)
    idx = select(cand)
    gate = (jnp.asarray(use_xyz) != 0).astype(jnp.float32)
    xyz_g = xyz * gate
    nx_g = new_xyz * gate
    gather = _make_gather(B, C, M, N)
    ofeat, ogxyz = gather(idx, feature, xyz_g, xyz, nx_g, new_xyz)
    group_feature = ofeat.reshape(B, C + 3, M, K)
    group_xyz = ogxyz.reshape(B, 3, M, K)
    return (group_feature, group_xyz)


# speedup vs baseline: 1072.2886x; 5.4163x over previous
import functools

import jax
import jax.numpy as jnp
from jax import lax
from jax.experimental import pallas as pl
from jax.experimental.pallas import tpu as pltpu
from jax.experimental.pallas import tpu_sc as plsc

RADIUS = 0.2
K = 64
NC = 2
NS = 16
L = 16


def _cand_body(nxyz_t_ref, xyz_ref, cand_ref):
    nx = nxyz_t_ref[0]
    p = xyz_ref[0]
    qx = nx[:, 0:1]
    qy = nx[:, 1:2]
    qz = nx[:, 2:3]
    px = p[0:1, :]
    py = p[1:2, :]
    pz = p[2:3, :]
    cross = jax.lax.dot_general(nx, p, (((1,), (0,)), ((), ())),
                                preferred_element_type=jnp.float32)
    q2 = (qx * qx + qy * qy) + qz * qz
    p2 = (px * px + py * py) + pz * pz
    d2 = (q2 + p2) - 2.0 * cross
    iota = jax.lax.broadcasted_iota(jnp.int32, d2.shape, 1)
    r2 = jnp.float32(RADIUS * RADIUS)
    cand_ref[0] = jnp.where(d2 < r2, iota, -1)


def _ball_query_cand(new_xyz, xyz):
    B, _, M = new_xyz.shape
    N = xyz.shape[2]
    Mb = 256
    nxyz_t = jnp.transpose(new_xyz, (0, 2, 1))
    grid = (B, M // Mb)
    return pl.pallas_call(
        _cand_body,
        grid=grid,
        in_specs=[
            pl.BlockSpec((1, Mb, 3), lambda b, i: (b, i, 0)),
            pl.BlockSpec((1, 3, N), lambda b, i: (b, 0, 0)),
        ],
        out_specs=pl.BlockSpec((1, Mb, N), lambda b, i: (b, i, 0)),
        out_shape=jax.ShapeDtypeStruct((B, M, N), jnp.int32),
    )(nxyz_t, xyz)


def _make_select(B, M, n):
    rows = B * M
    ngroups = n // L
    rpw = rows // (NC * NS)
    npairs = rpw // 2
    UNROLL = 8
    mesh = plsc.VectorSubcoreMesh(core_axis_name="c", subcore_axis_name="s")

    @functools.partial(
        pl.kernel,
        out_type=jax.ShapeDtypeStruct((rows * K,), jnp.int32),
        mesh=mesh,
        compiler_params=pltpu.CompilerParams(needs_layout_passes=False),
        scratch_types=[
            pltpu.VMEM((n,), jnp.int32),
            pltpu.VMEM((n,), jnp.int32),
            pltpu.VMEM((n + L,), jnp.int32),
            pltpu.VMEM((rpw * K,), jnp.int32),
            pltpu.SemaphoreType.DMA,
            pltpu.SemaphoreType.DMA,
        ],
    )
    def select(cand_hbm, idx_hbm, cbuf0, cbuf1, rowbuf, outbuf, sem0, sem1):
        cid = lax.axis_index("c")
        sid = lax.axis_index("s")
        wid = sid * NC + cid
        base = wid * rpw

        zeros16 = jnp.zeros((L,), jnp.int32)

        def start_row_copy(r, cbuf, sem):
            pltpu.async_copy(cand_hbm.at[r // M, r % M], cbuf, sem)

        def wait_row_copy(r, cbuf, sem):
            pltpu.make_async_copy(cand_hbm.at[r // M, r % M], cbuf, sem).wait()

        def scan_row(cbuf, r_local):
            def body(w, ptrv):
                v = cbuf[pl.ds(w * L, L)]
                msk = v >= 0
                cum = plsc.cumsum(msk.astype(jnp.int32))
                pos = ptrv + cum - 1
                mske = jnp.logical_and(msk, pos < K)
                plsc.store_scatter(rowbuf, [pos], v, mask=mske)
                return ptrv + plsc.all_reduce_population_count(msk)

            ptrv = plsc.parallel_loop(
                0, ngroups, 1, unroll=UNROLL, carry=zeros16)(body)
            v0 = rowbuf[pl.ds(0, L)]
            lanes0 = lax.iota(jnp.int32, L)
            firstv = plsc.cummax(jnp.where(lanes0 == 0, v0, jnp.int32(-2147483648)))
            fvec = jnp.where(ptrv > 0, firstv, 0)
            obase = r_local * K
            for g in range(K // L):
                cur = rowbuf[pl.ds(g * L, L)]
                lanes = lax.iota(jnp.int32, L) + (g * L)
                outbuf[pl.ds(obase + g * L, L)] = jnp.where(lanes < ptrv, cur, fvec)

        start_row_copy(base, cbuf0, sem0)

        def pair(j, _):
            r0 = base + 2 * j
            start_row_copy(r0 + 1, cbuf1, sem1)
            wait_row_copy(r0, cbuf0, sem0)
            scan_row(cbuf0, 2 * j)

            @pl.when(j < npairs - 1)
            def _():
                start_row_copy(r0 + 2, cbuf0, sem0)

            wait_row_copy(r0 + 1, cbuf1, sem1)
            scan_row(cbuf1, 2 * j + 1)
            return 0

        lax.fori_loop(0, npairs, pair, 0)
        pltpu.sync_copy(outbuf, idx_hbm.at[pl.ds(base * K, rpw * K)])

    return select


def _make_gather(B, C, M, N):
    MK = M * K
    HALF = MK // 2
    CTOT = C + 6
    NSLAB = 2 * B
    SPS = (NC * NS) // NSLAB
    CPS = (CTOT + SPS - 1) // SPS
    CHUNK = 4096
    NCHUNK = HALF // CHUNK
    NGRP = CHUNK // L
    NBUF = 4
    mesh = plsc.VectorSubcoreMesh(core_axis_name="c", subcore_axis_name="s")

    @functools.partial(
        pl.kernel,
        out_type=(
            jax.ShapeDtypeStruct((B * (C + 3) * MK,), jnp.float32),
            jax.ShapeDtypeStruct((B * 3 * MK,), jnp.float32),
        ),
        mesh=mesh,
        compiler_params=pltpu.CompilerParams(needs_layout_passes=False),
        scratch_types=[
            pltpu.VMEM((HALF,), jnp.int32),
            pltpu.VMEM((N,), jnp.float32),
            pltpu.VMEM((M,), jnp.float32),
            [pltpu.VMEM((CHUNK,), jnp.float32) for _ in range(NBUF)],
            [pltpu.SemaphoreType.DMA for _ in range(NBUF)],
            pltpu.SemaphoreType.DMA,
        ],
    )
    def gather(idx_hbm, feat_hbm, xyzg_hbm, xyz_hbm, nxg_hbm, nx_hbm,
               ofeat_hbm, ogxyz_hbm, ibuf, tbl, nxtbl, obufs, osems, sem0):
        cid = lax.axis_index("c")
        sid = lax.axis_index("s")
        wid = sid * NC + cid
        slab = wid // SPS
        lane = wid % SPS
        b = slab // 2
        h = slab % 2
        slab_off = h * HALF
        lanes0 = lax.iota(jnp.int32, L)

        pltpu.sync_copy(idx_hbm.at[pl.ds(b * MK + slab_off, HALF)], ibuf)

        def do_channel(ci, _):
            c_glob = lane * CPS + ci

            @pl.when(c_glob < CTOT)
            def _():
                is_feat = c_glob < C
                is_fxyz = jnp.logical_and(c_glob >= C, c_glob < C + 3)
                is_gxyz = c_glob >= C + 3
                cf = jnp.minimum(c_glob, C - 1)
                cx = jnp.clip(c_glob - C, 0, 2)
                cg = jnp.clip(c_glob - (C + 3), 0, 2)
                oc = jnp.minimum(c_glob, C + 2)

                @pl.when(is_feat)
                def _():
                    pltpu.sync_copy(feat_hbm.at[b, cf], tbl)

                @pl.when(is_fxyz)
                def _():
                    pltpu.sync_copy(xyzg_hbm.at[b, cx], tbl)
                    pltpu.sync_copy(nxg_hbm.at[b, cx], nxtbl)

                @pl.when(is_gxyz)
                def _():
                    pltpu.sync_copy(xyz_hbm.at[b, cg], tbl)
                    pltpu.sync_copy(nx_hbm.at[b, cg], nxtbl)

                def fill(obuf, ch):
                    base = ch * CHUNK

                    def grp(i):
                        o = i * L
                        iv = ibuf[pl.ds(base + o, L)]
                        g = plsc.load_gather(tbl, [iv])
                        obuf[pl.ds(o, L)] = g

                    def grp_sub(i):
                        o = i * L
                        iv = ibuf[pl.ds(base + o, L)]
                        g = plsc.load_gather(tbl, [iv])
                        mv = (slab_off + base + o + lanes0) // K
                        nxv = plsc.load_gather(nxtbl, [mv])
                        obuf[pl.ds(o, L)] = g - nxv

                    @pl.when(is_feat)
                    def _():
                        plsc.parallel_loop(0, NGRP, 1, unroll=8)(grp)

                    @pl.when(jnp.logical_not(is_feat))
                    def _():
                        plsc.parallel_loop(0, NGRP, 1, unroll=8)(grp_sub)

                def flush(obuf, ch, sem):
                    dst_off = slab_off + ch * CHUNK
                    feat_at = (b * (C + 3) + oc) * MK + dst_off
                    gxyz_at = (b * 3 + cg) * MK + dst_off

                    @pl.when(jnp.logical_not(is_gxyz))
                    def _():
                        pltpu.async_copy(
                            obuf, ofeat_hbm.at[pl.ds(feat_at, CHUNK)], sem)

                    @pl.when(is_gxyz)
                    def _():
                        pltpu.async_copy(
                            obuf, ogxyz_hbm.at[pl.ds(gxyz_at, CHUNK)], sem)

                def wait_flush(obuf, ch, sem):
                    dst_off = slab_off + ch * CHUNK
                    feat_at = (b * (C + 3) + oc) * MK + dst_off
                    gxyz_at = (b * 3 + cg) * MK + dst_off

                    @pl.when(jnp.logical_not(is_gxyz))
                    def _():
                        pltpu.make_async_copy(
                            obuf, ofeat_hbm.at[pl.ds(feat_at, CHUNK)],
                            sem).wait()

                    @pl.when(is_gxyz)
                    def _():
                        pltpu.make_async_copy(
                            obuf, ogxyz_hbm.at[pl.ds(gxyz_at, CHUNK)],
                            sem).wait()

                def ring(j, _):
                    for s in range(NBUF):
                        ch = j * NBUF + s

                        @pl.when(j > 0)
                        def _():
                            wait_flush(obufs[s], ch - NBUF, osems[s])

                        fill(obufs[s], ch)
                        flush(obufs[s], ch, osems[s])
                    return 0

                lax.fori_loop(0, NCHUNK // NBUF, ring, 0)
                for s in range(NBUF):
                    wait_flush(obufs[s], NCHUNK - NBUF + s, osems[s])

            return 0

        lax.fori_loop(0, CPS, do_channel, 0)

    return gather


def kernel(new_xyz, xyz, feature, use_xyz):
    B, _, M = new_xyz.shape
    C = feature.shape[1]
    N = xyz.shape[2]
    cand = _ball_query_cand(new_xyz, xyz)
    select = _make_select(B, M, N)
    idx = select(cand)
    gate = (jnp.asarray(use_xyz) != 0).astype(jnp.float32)
    xyz_g = xyz * gate
    nx_g = new_xyz * gate
    gather = _make_gather(B, C, M, N)
    ofeat, ogxyz = gather(idx, feature, xyz_g, xyz, nx_g, new_xyz)
    group_feature = ofeat.reshape(B, C + 3, M, K)
    group_xyz = ogxyz.reshape(B, 3, M, K)
    return (group_feature, group_xyz)
